# Initial kernel scaffold; baseline (speedup 1.0000x reference)
import jax
import jax.numpy as jnp
from jax.experimental import pallas as pl


def kernel(x, tables):
    raise NotImplementedError("write your pallas kernel here")


# speedup vs baseline: 19.4304x; 19.4304x over previous
import numpy as np
import jax
import jax.numpy as jnp
from jax import lax
from jax.experimental import pallas as pl
from jax.experimental.pallas import tpu as pltpu
from jax.experimental.pallas import tpu_sc as plsc

_T = 1 << 19
_L = 16
_F = 2
_N = 262144
_B = np.exp((np.log(np.float32(512.0)) - np.log(np.float32(16.0))) / np.float32(15.0)).astype(np.float32)
_RES = [float(np.floor(np.float32(16.0) * (_B ** np.float32(l)))) for l in range(_L)]
_K1 = np.uint32(2654435761).astype(np.int32).item()
_K2 = 805459861

_NW = 32
_PPW = _N // _NW
_G = _PPW // 16


def _body(xt_hbm, tab_hbm, out_hbm, mask_hbm, xv, idx_ref, rows_ref, out_buf, mask_buf, sem):
    wid = lax.axis_index("s") * 2 + lax.axis_index("c")
    base = wid * _PPW
    for d in range(3):
        pltpu.sync_copy(xt_hbm.at[pl.ds(d * _N + base, _PPW)],
                        xv.at[pl.ds(d * _PPW, _PPW)])

    def group(g, carry):
        r0 = g * 16
        x0 = xv[pl.ds(r0, 16)]
        x1 = xv[pl.ds(_PPW + r0, 16)]
        x2 = xv[pl.ds(2 * _PPW + r0, 16)]
        xn0 = (x0 + 1.0) * 0.5
        xn1 = (x1 + 1.0) * 0.5
        xn2 = (x2 + 1.0) * 0.5

        ws = []
        copies = []
        for l in range(_L):
            nl = _RES[l]
            t0 = xn0 * nl
            t1 = xn1 * nl
            t2 = xn2 * nl
            m0 = t0.astype(jnp.int32)
            m1 = t1.astype(jnp.int32)
            m2 = t2.astype(jnp.int32)
            ws.append((t0 - m0.astype(jnp.float32),
                       t1 - m1.astype(jnp.float32),
                       t2 - m2.astype(jnp.float32)))
            a = (m0, m0 + 1)
            b0 = m1 * _K1
            b = (b0, b0 + _K1)
            c0 = m2 * _K2
            c = (c0, c0 + _K2)
            for ci in range(8):
                h = ((a[ci >> 2] ^ b[(ci >> 1) & 1] ^ c[ci & 1]) & (_T - 1))
                flat = (h + l * _T) * 2
                idx_ref[pl.ds(l * 256 + ci * 16, 16)] = flat
                idx_ref[pl.ds(l * 256 + 128 + ci * 16, 16)] = flat + 1
            for f in range(_F):
                o = l * 256 + f * 128
                copies.append(pltpu.async_copy(
                    tab_hbm.at[idx_ref.at[pl.ds(o, 128)]],
                    rows_ref.at[pl.ds(o, 128)], sem))

        km = ((x0 >= -1.0) & (x0 <= 1.0) & (x1 >= -1.0) & (x1 <= 1.0)
              & (x2 >= -1.0) & (x2 <= 1.0))
        mask_buf[...] = jnp.where(km, 1, 0).astype(jnp.int32)

        for l in range(_L):
            copies[2 * l].wait()
            copies[2 * l + 1].wait()
            w0, w1, w2 = ws[l]
            u0 = 1.0 - w0
            u1 = 1.0 - w1
            u2 = 1.0 - w2
            for f in range(_F):
                o = l * 256 + f * 128
                ve = [rows_ref[pl.ds(o + ci * 16, 16)] for ci in range(8)]
                c00 = ve[0] * u0 + ve[4] * w0
                c01 = ve[1] * u0 + ve[5] * w0
                c10 = ve[2] * u0 + ve[6] * w0
                c11 = ve[3] * u0 + ve[7] * w0
                d0 = c00 * u1 + c10 * w1
                d1 = c01 * u1 + c11 * w1
                res = d0 * u2 + d1 * w2
                out_buf[pl.ds((2 * l + f) * 16, 16)] = res

        pltpu.sync_copy(out_buf, out_hbm.at[pl.ds((base + r0) * 2 * _L, 2 * _L * 16)])
        pltpu.sync_copy(mask_buf, mask_hbm.at[pl.ds(base + r0, 16)])
        return carry

    lax.fori_loop(0, _G, group, 0)


_call = pl.kernel(
    _body,
    out_type=[
        jax.ShapeDtypeStruct((_N * 2 * _L,), jnp.float32),
        jax.ShapeDtypeStruct((_N,), jnp.int32),
    ],
    mesh=plsc.VectorSubcoreMesh(core_axis_name="c", subcore_axis_name="s"),
    scratch_types=[
        pltpu.VMEM((3 * _PPW,), jnp.float32),
        pltpu.VMEM((_L * 256,), jnp.int32),
        pltpu.VMEM((_L * 256,), jnp.float32),
        pltpu.VMEM((2 * _L * 16,), jnp.float32),
        pltpu.VMEM((16,), jnp.int32),
        pltpu.SemaphoreType.DMA,
    ],
)


def kernel(x, tables):
    xt = x.T.reshape(-1)
    tab = tables.reshape(-1)
    outg, mask = _call(xt, tab)
    out = outg.reshape(_N // 16, 2 * _L, 16).transpose(0, 2, 1).reshape(_N, 2 * _L)
    return out, mask.astype(bool)
